# baseline (speedup 1.0000x reference)
import jax
import jax.numpy as jnp
from jax.experimental import pallas as pl
from jax.experimental.pallas import tpu as pltpu

_IN = 4096
_TB = 1024


def _matvec_relu_kernel(x_ref, w_ref, b_ref, o_ref):
    w_rep = jnp.broadcast_to(w_ref[...], (128, _IN))
    y = jax.lax.dot_general(
        x_ref[...], w_rep,
        dimension_numbers=(((1,), (1,)), ((), ())),
        preferred_element_type=jnp.float32,
    )
    o_ref[...] = jnp.maximum(y[:, 0:1] + b_ref[0, 0], 0.0).astype(o_ref.dtype)


def kernel(x, weight, bias):
    B = x.shape[0]
    assert x.shape[1] == _IN

    if B <= _TB:
        tb, num_tiles = B, 1
    else:
        tb = _TB
        num_tiles = pl.cdiv(B, tb)

    bias_smem = jnp.asarray(bias, jnp.float32).reshape(1, 1)

    out = pl.pallas_call(
        _matvec_relu_kernel,
        out_shape=jax.ShapeDtypeStruct((num_tiles * tb, 1), x.dtype),
        grid=(num_tiles,),
        in_specs=[
            pl.BlockSpec((tb, _IN), lambda i: (i, 0)),
            pl.BlockSpec((1, _IN), lambda i: (0, 0)),
            pl.BlockSpec(memory_space=pltpu.MemorySpace.SMEM),
        ],
        out_specs=pl.BlockSpec((tb, 1), lambda i: (i, 0)),
        compiler_params=pltpu.CompilerParams(
            dimension_semantics=("parallel",),
            vmem_limit_bytes=48 << 20,
        ),
    )(x, weight, bias_smem)

    return out[:B]


# speedup vs baseline: 1.1147x; 1.1147x over previous
import jax
import jax.numpy as jnp
from jax.experimental import pallas as pl
from jax.experimental.pallas import tpu as pltpu

_IN = 4096
_TB = 1024


def _matvec_relu_kernel(x_ref, w_ref, b_ref, o_ref):
    w_rep = jnp.broadcast_to(w_ref[...], (128, _IN))
    y = jax.lax.dot_general(
        w_rep, x_ref[...],
        dimension_numbers=(((1,), (1,)), ((), ())),
        preferred_element_type=jnp.float32,
    )
    o_ref[...] = jnp.maximum(y[0:1, :] + b_ref[0, 0], 0.0).astype(o_ref.dtype)


def kernel(x, weight, bias):
    B = x.shape[0]
    assert x.shape[1] == _IN

    if B <= _TB:
        tb, num_tiles = B, 1
    else:
        tb = _TB
        num_tiles = pl.cdiv(B, tb)

    bias_smem = jnp.asarray(bias, jnp.float32).reshape(1, 1)

    out = pl.pallas_call(
        _matvec_relu_kernel,
        out_shape=jax.ShapeDtypeStruct((1, num_tiles * tb), x.dtype),
        grid=(num_tiles,),
        in_specs=[
            pl.BlockSpec((tb, _IN), lambda i: (i, 0)),
            pl.BlockSpec((1, _IN), lambda i: (0, 0)),
            pl.BlockSpec(memory_space=pltpu.MemorySpace.SMEM),
        ],
        out_specs=pl.BlockSpec((1, tb), lambda i: (0, i)),
        compiler_params=pltpu.CompilerParams(
            dimension_semantics=("parallel",),
            vmem_limit_bytes=48 << 20,
        ),
    )(x, weight, bias_smem)

    return out[0, :B].reshape(B, 1)
